# baseline (speedup 1.0000x reference)
import jax
import jax.numpy as jnp
from jax import lax
from jax.experimental import pallas as pl
from jax.experimental.pallas import tpu as pltpu
from jax.experimental.pallas import tpu_sc as plsc

_BATCH = 2
_SEQ = 8192
_EMBED = 1024

_NC = 2
_NS = 16
_NW = _NC * _NS
_L = 16

_ROWS_PER_W = _SEQ // _NW
_R = 8
_NCHUNK = _ROWS_PER_W // _R


def _sc_add_body(x_hbm, pos_hbm, out_hbm,
                 x0a, x0b, x1a, x1b, y0a, y0b, y1a, y1b, pba, pbb,
                 sx0a, sx0b, sx1a, sx1b, sy0a, sy0b, sy1a, sy1b, spa, spb):
    x0 = (x0a, x0b)
    x1 = (x1a, x1b)
    y0 = (y0a, y0b)
    y1 = (y1a, y1b)
    pb = (pba, pbb)
    sx0 = (sx0a, sx0b)
    sx1 = (sx1a, sx1b)
    sy0 = (sy0a, sy0b)
    sy1 = (sy1a, sy1b)
    sp = (spa, spb)

    wid = lax.axis_index("s") * _NC + lax.axis_index("c")
    row_base = wid * _ROWS_PER_W

    def loads(ci, j):
        r0 = row_base + ci * _R
        return (
            pltpu.make_async_copy(pos_hbm.at[pl.ds(r0, _R), :], pb[j], sp[j]),
            pltpu.make_async_copy(x_hbm.at[pl.ds(r0, _R), :], x0[j], sx0[j]),
            pltpu.make_async_copy(x_hbm.at[pl.ds(_SEQ + r0, _R), :],
                                  x1[j], sx1[j]),
        )

    def stores(ci, j):
        r0 = row_base + ci * _R
        return (
            pltpu.make_async_copy(y0[j], out_hbm.at[pl.ds(r0, _R), :], sy0[j]),
            pltpu.make_async_copy(y1[j], out_hbm.at[pl.ds(_SEQ + r0, _R), :],
                                  sy1[j]),
        )

    for c in loads(0, 0):
        c.start()
    for c in loads(1, 1):
        c.start()

    def step(p, carry):
        for j in (0, 1):
            ci = 2 * p + j
            for c in loads(ci, j):
                c.wait()

            @pl.when(ci >= 2)
            def _():
                for c in stores(ci - 2, j):
                    c.wait()

            x0j, x1j, y0j, y1j, pbj = x0[j], x1[j], y0[j], y1[j], pb[j]

            @plsc.parallel_loop(0, _R, step=1, unroll=1)
            def _(r):
                for t in range(_EMBED // _L):
                    cs = pl.ds(t * _L, _L)
                    pv = pbj[r, cs]
                    y0j[r, cs] = x0j[r, cs] + pv
                    y1j[r, cs] = x1j[r, cs] + pv

            for c in stores(ci, j):
                c.start()

            @pl.when(ci + 2 < _NCHUNK)
            def _():
                for c in loads(ci + 2, j):
                    c.start()
        return carry

    lax.fori_loop(0, _NCHUNK // 2, step, 0)

    for c in stores(_NCHUNK - 2, 0):
        c.wait()
    for c in stores(_NCHUNK - 1, 1):
        c.wait()


_sc_add = pl.kernel(
    _sc_add_body,
    out_type=jax.ShapeDtypeStruct((_BATCH * _SEQ, _EMBED), jnp.float32),
    mesh=plsc.VectorSubcoreMesh(core_axis_name="c", subcore_axis_name="s"),
    compiler_params=pltpu.CompilerParams(use_tc_tiling_on_sc=True, skip_device_barrier=True, disable_bounds_checks=True, disable_semaphore_checks=True),
    scratch_types=(
        [pltpu.VMEM((_R, _EMBED), jnp.float32)] * 10
        + [pltpu.SemaphoreType.DMA] * 10
    ),
)


def kernel(x, position_matrix):
    out2d = _sc_add(x.reshape(_BATCH * _SEQ, _EMBED), position_matrix)
    return out2d.reshape(x.shape)


# speedup vs baseline: 1.0951x; 1.0951x over previous
import jax
import jax.numpy as jnp
from jax import lax
from jax.experimental import pallas as pl
from jax.experimental.pallas import tpu as pltpu
from jax.experimental.pallas import tpu_sc as plsc

_BATCH = 2
_SEQ = 8192
_EMBED = 1024

_NC = 2
_NS = 16
_NW = _NC * _NS
_L = 16

_ROWS_PER_W = _SEQ // _NW
_R = 8
_NCHUNK = _ROWS_PER_W // _R


def _sc_add_body(x_hbm, pos_hbm, out_hbm,
                 x0a, x0b, x1a, x1b, y0a, y0b, y1a, y1b, pba, pbb,
                 sx0a, sx0b, sx1a, sx1b, sy0a, sy0b, sy1a, sy1b, spa, spb):
    x0 = (x0a, x0b)
    x1 = (x1a, x1b)
    y0 = (y0a, y0b)
    y1 = (y1a, y1b)
    pb = (pba, pbb)
    sx0 = (sx0a, sx0b)
    sx1 = (sx1a, sx1b)
    sy0 = (sy0a, sy0b)
    sy1 = (sy1a, sy1b)
    sp = (spa, spb)

    wid = lax.axis_index("s") * _NC + lax.axis_index("c")
    row_base = wid * _ROWS_PER_W

    def loads(ci, j):
        r0 = row_base + ci * _R
        return (
            pltpu.make_async_copy(pos_hbm.at[pl.ds(r0, _R), :], pb[j], sp[j]),
            pltpu.make_async_copy(x_hbm.at[pl.ds(r0, _R), :], x0[j], sx0[j]),
            pltpu.make_async_copy(x_hbm.at[pl.ds(_SEQ + r0, _R), :],
                                  x1[j], sx1[j]),
        )

    def stores(ci, j):
        r0 = row_base + ci * _R
        return (
            pltpu.make_async_copy(y0[j], out_hbm.at[pl.ds(r0, _R), :], sy0[j]),
            pltpu.make_async_copy(y1[j], out_hbm.at[pl.ds(_SEQ + r0, _R), :],
                                  sy1[j]),
        )

    for c in loads(0, 0):
        c.start()
    for c in loads(1, 1):
        c.start()

    def step(p, carry):
        for j in (0, 1):
            ci = 2 * p + j
            for c in loads(ci, j):
                c.wait()

            @pl.when(ci >= 2)
            def _():
                for c in stores(ci - 2, j):
                    c.wait()

            x0j, x1j, y0j, y1j, pbj = x0[j], x1[j], y0[j], y1[j], pb[j]

            @plsc.parallel_loop(0, _R, step=1, unroll=1)
            def _(r):
                @plsc.parallel_loop(0, _EMBED, step=_L, unroll=8)
                def _(t):
                    cs = pl.ds(t, _L)
                    pv = pbj[r, cs]
                    y0j[r, cs] = x0j[r, cs] + pv
                    y1j[r, cs] = x1j[r, cs] + pv

            for c in stores(ci, j):
                c.start()

            @pl.when(ci + 2 < _NCHUNK)
            def _():
                for c in loads(ci + 2, j):
                    c.start()
        return carry

    lax.fori_loop(0, _NCHUNK // 2, step, 0)

    for c in stores(_NCHUNK - 2, 0):
        c.wait()
    for c in stores(_NCHUNK - 1, 1):
        c.wait()


_sc_add = pl.kernel(
    _sc_add_body,
    out_type=jax.ShapeDtypeStruct((_BATCH * _SEQ, _EMBED), jnp.float32),
    mesh=plsc.VectorSubcoreMesh(core_axis_name="c", subcore_axis_name="s"),
    compiler_params=pltpu.CompilerParams(use_tc_tiling_on_sc=True, skip_device_barrier=True, disable_bounds_checks=True, disable_semaphore_checks=True),
    scratch_types=(
        [pltpu.VMEM((_R, _EMBED), jnp.float32)] * 10
        + [pltpu.SemaphoreType.DMA] * 10
    ),
)


def kernel(x, position_matrix):
    out2d = _sc_add(x.reshape(_BATCH * _SEQ, _EMBED), position_matrix)
    return out2d.reshape(x.shape)
